# baseline (speedup 1.0000x reference)
import functools

import jax
import jax.numpy as jnp
from jax import lax
from jax.experimental import pallas as pl
from jax.experimental.pallas import tpu as pltpu
from jax.experimental.pallas import tpu_sc as plsc

N = 10000
E = 320000
NT = 16
NC = 2
NW = NC * NT
CHUNK = 128
KB = 40
NB1 = 2
NB2 = 4
E_PAD = NW * NB1 * KB * CHUNK
NBLK = E_PAD // (KB * CHUNK)
NP = 10112
ZPT = NP // NT


def _make_seg_sum(nb, split_edges):
    mesh = plsc.VectorSubcoreMesh(core_axis_name="c", subcore_axis_name="s")

    @functools.partial(
        pl.kernel,
        out_type=jax.ShapeDtypeStruct((NC * NP, 128), jnp.float32),
        mesh=mesh,
        scratch_types=[
            pltpu.VMEM((KB, CHUNK), jnp.int32),
            pltpu.VMEM((KB, CHUNK), jnp.int32),
            pltpu.VMEM((2, CHUNK, 128), jnp.float32),
            pltpu.VMEM_SHARED((NP, 128), jnp.float32),
            pltpu.SemaphoreType.DMA,
            pltpu.SemaphoreType.DMA,
            pltpu.SemaphoreType.DMA,
            pltpu.SemaphoreType.DMA,
        ],
    )
    def seg_sum(table, srcs, dsts, zeros, out, src_v, dst_v, rows_v, acc,
                gsem0, gsem1, ssem0, ssem1):
        cid = lax.axis_index("c")
        sid = lax.axis_index("s")
        if split_edges:
            src_base = (cid * NT + sid) * nb
        else:
            src_base = sid * nb
        dst_base = src_base
        pltpu.sync_copy(zeros, acc.at[pl.ds(sid * ZPT, ZPT)])
        plsc.subcore_barrier()

        gsems = (gsem0, gsem1)
        ssems = (ssem0, ssem1)

        def blk(b, carry):
            pltpu.sync_copy(srcs.at[src_base + b], src_v)
            pltpu.sync_copy(dsts.at[dst_base + b], dst_v)
            if not split_edges:
                off = cid * NP

                def add_off(r, c2):
                    for c8 in range(CHUNK // 16):
                        sl = pl.ds(c8 * 16, 16)
                        src_v[r, sl] = src_v[r, sl] + off
                    return c2

                lax.fori_loop(0, KB, add_off, 0, unroll=False)
            gd = [None, None]
            sd = [None, None]
            gd[0] = pltpu.async_copy(table.at[src_v.at[0]], rows_v.at[0], gsems[0])
            for k in range(KB):
                s = k % 2
                gd[s].wait()
                sd[s] = pltpu.async_copy(rows_v.at[s], acc.at[dst_v.at[k]],
                                         ssems[s], add=True)
                if k + 1 < KB:
                    if sd[1 - s] is not None:
                        sd[1 - s].wait()
                    gd[1 - s] = pltpu.async_copy(
                        table.at[src_v.at[k + 1]], rows_v.at[1 - s], gsems[1 - s])
            sd[(KB - 2) % 2].wait()
            sd[(KB - 1) % 2].wait()
            return carry

        lax.fori_loop(0, nb, blk, 0, unroll=False)
        plsc.subcore_barrier()
        pltpu.sync_copy(acc.at[pl.ds(sid * ZPT, ZPT)],
                        out.at[pl.ds(cid * NP + sid * ZPT, ZPT)])

    return seg_sum


_seg1 = _make_seg_sum(NB1, True)
_seg2 = _make_seg_sum(NB2, False)

_BR = 632


def _mm1_body(a0, a1, w, o):
    ax = a0[...] + a1[...]
    o[...] = jnp.maximum(jnp.dot(ax, w[...], preferred_element_type=jnp.float32), 0.0)


_NRB = NP // _BR


def _mm1(a0, a1, w0):
    grid = (2, _NRB)
    return pl.pallas_call(
        _mm1_body,
        grid=grid,
        in_specs=[
            pl.BlockSpec((_BR, 128), lambda c, i: (i, 0)),
            pl.BlockSpec((_BR, 128), lambda c, i: (i, 0)),
            pl.BlockSpec((128, 128), lambda c, i: (0, c)),
        ],
        out_specs=pl.BlockSpec((_BR, 128), lambda c, i: (c * _NRB + i, 0)),
        out_shape=jax.ShapeDtypeStruct((2 * NP, 128), jnp.float32),
    )(a0, a1, w0)


def _mm2_body(a0, a1, w, o):
    ah = jnp.concatenate([a0[...], a1[...]], axis=1)
    o[...] = jnp.dot(ah, w[...], preferred_element_type=jnp.float32)


def _mm2(a0, a1, w12):
    grid = (pl.cdiv(N, _BR),)
    return pl.pallas_call(
        _mm2_body,
        grid=grid,
        in_specs=[
            pl.BlockSpec((_BR, 128), lambda i: (i, 0)),
            pl.BlockSpec((_BR, 128), lambda i: (i, 0)),
            pl.BlockSpec((256, 256), lambda i: (0, 0)),
        ],
        out_specs=pl.BlockSpec((_BR, 256), lambda i: (i, 0)),
        out_shape=jax.ShapeDtypeStruct((N, 256), jnp.float32),
    )(a0, a1, w12)


def kernel(x, edge_index, W0, W1, W2):
    src = edge_index[0]
    dst = edge_index[1]
    zeros = jnp.zeros((ZPT, 128), jnp.float32)

    pad = E_PAD - E
    pad_src = jnp.arange(pad, dtype=jnp.int32) % N
    pad_dst = N + (jnp.arange(pad, dtype=jnp.int32) % (NP - N))
    srcp = jnp.concatenate([src, pad_src]).reshape(NBLK, KB, CHUNK)
    dstp = jnp.concatenate([dst, pad_dst]).reshape(NBLK, KB, CHUNK)

    xp = jnp.concatenate([x, jnp.zeros((NP - N, 128), jnp.float32)], axis=0)
    ax = _seg1(xp, srcp, dstp, zeros)
    h = _mm1(ax[:NP], ax[NP:], W0)
    ah = _seg2(h, srcp, dstp, zeros)
    out = _mm2(ah[:NP], ah[NP:], jnp.concatenate([W1, W2], axis=1))
    return out


# speedup vs baseline: 1.0283x; 1.0283x over previous
import functools

import jax
import jax.numpy as jnp
from jax import lax
from jax.experimental import pallas as pl
from jax.experimental.pallas import tpu as pltpu
from jax.experimental.pallas import tpu_sc as plsc

N = 10000
E = 320000
NT = 16
NC = 2
NW = NC * NT
CHUNK = 128
KB = 40
NB1 = 2
NB2 = 4
E_PAD = NW * NB1 * KB * CHUNK
NBLK = E_PAD // (KB * CHUNK)
NP = 10112
ZPT = NP // NT


def _make_seg_sum(nb, split_edges):
    mesh = plsc.VectorSubcoreMesh(core_axis_name="c", subcore_axis_name="s")

    @functools.partial(
        pl.kernel,
        out_type=jax.ShapeDtypeStruct((NC * NP, 128), jnp.float32),
        mesh=mesh,
        scratch_types=[
            pltpu.VMEM((KB, CHUNK), jnp.int32),
            pltpu.VMEM((KB, CHUNK), jnp.int32),
            pltpu.VMEM((2, CHUNK, 128), jnp.float32),
            pltpu.VMEM_SHARED((NP, 128), jnp.float32),
            pltpu.SemaphoreType.DMA,
            pltpu.SemaphoreType.DMA,
            pltpu.SemaphoreType.DMA,
            pltpu.SemaphoreType.DMA,
        ],
    )
    def seg_sum(table, srcs, dsts, zeros, out, src_v, dst_v, rows_v, acc,
                gsem0, gsem1, ssem0, ssem1):
        cid = lax.axis_index("c")
        sid = lax.axis_index("s")
        if split_edges:
            src_base = (cid * NT + sid) * nb
            dst_base = src_base
        else:
            src_base = (cid * NT + sid) * nb
            dst_base = sid * nb
        pltpu.sync_copy(zeros, acc.at[pl.ds(sid * ZPT, ZPT)])
        plsc.subcore_barrier()

        gsems = (gsem0, gsem1)
        ssems = (ssem0, ssem1)

        def blk(b, carry):
            pltpu.sync_copy(srcs.at[src_base + b], src_v)
            pltpu.sync_copy(dsts.at[dst_base + b], dst_v)
            gd = [None, None]
            sd = [None, None]
            gd[0] = pltpu.async_copy(table.at[src_v.at[0]], rows_v.at[0], gsems[0])
            for k in range(KB):
                s = k % 2
                gd[s].wait()
                sd[s] = pltpu.async_copy(rows_v.at[s], acc.at[dst_v.at[k]],
                                         ssems[s], add=True)
                if k + 1 < KB:
                    if sd[1 - s] is not None:
                        sd[1 - s].wait()
                    gd[1 - s] = pltpu.async_copy(
                        table.at[src_v.at[k + 1]], rows_v.at[1 - s], gsems[1 - s])
            sd[(KB - 2) % 2].wait()
            sd[(KB - 1) % 2].wait()
            return carry

        lax.fori_loop(0, nb, blk, 0, unroll=False)
        plsc.subcore_barrier()
        pltpu.sync_copy(acc.at[pl.ds(sid * ZPT, ZPT)],
                        out.at[pl.ds(cid * NP + sid * ZPT, ZPT)])

    return seg_sum


_seg1 = _make_seg_sum(NB1, True)
_seg2 = _make_seg_sum(NB2, False)

_BR = 1264


def _mm1_body(a0, a1, w, o):
    ax = a0[...] + a1[...]
    o[...] = jnp.maximum(jnp.dot(ax, w[...], preferred_element_type=jnp.float32), 0.0)


_NRB = NP // _BR


def _mm1(a0, a1, w0):
    grid = (2, _NRB)
    return pl.pallas_call(
        _mm1_body,
        grid=grid,
        in_specs=[
            pl.BlockSpec((_BR, 128), lambda c, i: (i, 0)),
            pl.BlockSpec((_BR, 128), lambda c, i: (i, 0)),
            pl.BlockSpec((128, 128), lambda c, i: (0, c)),
        ],
        out_specs=pl.BlockSpec((_BR, 128), lambda c, i: (c * _NRB + i, 0)),
        out_shape=jax.ShapeDtypeStruct((2 * NP, 128), jnp.float32),
    )(a0, a1, w0)


def _mm2_body(a0, a1, w, o):
    ah = jnp.concatenate([a0[...], a1[...]], axis=1)
    o[...] = jnp.dot(ah, w[...], preferred_element_type=jnp.float32)


def _mm2(a0, a1, w12):
    grid = (pl.cdiv(N, _BR),)
    return pl.pallas_call(
        _mm2_body,
        grid=grid,
        in_specs=[
            pl.BlockSpec((_BR, 128), lambda i: (i, 0)),
            pl.BlockSpec((_BR, 128), lambda i: (i, 0)),
            pl.BlockSpec((256, 256), lambda i: (0, 0)),
        ],
        out_specs=pl.BlockSpec((_BR, 256), lambda i: (i, 0)),
        out_shape=jax.ShapeDtypeStruct((N, 256), jnp.float32),
    )(a0, a1, w12)


def kernel(x, edge_index, W0, W1, W2):
    src = edge_index[0]
    dst = edge_index[1]
    zeros = jnp.zeros((ZPT, 128), jnp.float32)

    pad = E_PAD - E
    pad_src = jnp.arange(pad, dtype=jnp.int32) % N
    pad_dst = N + (jnp.arange(pad, dtype=jnp.int32) % (NP - N))
    srcp = jnp.concatenate([src, pad_src]).reshape(NBLK, KB, CHUNK)
    dstp = jnp.concatenate([dst, pad_dst]).reshape(NBLK, KB, CHUNK)
    src2 = jnp.concatenate([srcp, srcp + NP], axis=0)

    xp = jnp.concatenate([x, jnp.zeros((NP - N, 128), jnp.float32)], axis=0)
    ax = _seg1(xp, srcp, dstp, zeros)
    h = _mm1(ax[:NP], ax[NP:], W0)
    ah = _seg2(h, src2, dstp, zeros)
    out = _mm2(ah[:NP], ah[NP:], jnp.concatenate([W1, W2], axis=1))
    return out


# speedup vs baseline: 1.0462x; 1.0174x over previous
import functools

import jax
import jax.numpy as jnp
from jax import lax
from jax.experimental import pallas as pl
from jax.experimental.pallas import tpu as pltpu
from jax.experimental.pallas import tpu_sc as plsc

N = 10000
E = 320000
NT = 16
NC = 2
NW = NC * NT
CHUNK = 128
KB = 40
NB1 = 2
NB2 = 4
E_PAD = NW * NB1 * KB * CHUNK
NBLK = E_PAD // (KB * CHUNK)
NP = 10112
ZPT = NP // NT


def _make_seg_sum(nb, split_edges):
    mesh = plsc.VectorSubcoreMesh(core_axis_name="c", subcore_axis_name="s")

    @functools.partial(
        pl.kernel,
        out_type=jax.ShapeDtypeStruct((NC * NP, 128), jnp.float32),
        mesh=mesh,
        scratch_types=[
            pltpu.VMEM((KB, CHUNK), jnp.int32),
            pltpu.VMEM((KB, CHUNK), jnp.int32),
            pltpu.VMEM((2, CHUNK, 128), jnp.float32),
            pltpu.VMEM_SHARED((NP, 128), jnp.float32),
            pltpu.SemaphoreType.DMA,
            pltpu.SemaphoreType.DMA,
            pltpu.SemaphoreType.DMA,
            pltpu.SemaphoreType.DMA,
        ],
    )
    def seg_sum(table, srcs, dsts, zeros, out, src_v, dst_v, rows_v, acc,
                gsem0, gsem1, ssem0, ssem1):
        cid = lax.axis_index("c")
        sid = lax.axis_index("s")
        if split_edges:
            src_base = (cid * NT + sid) * nb
            dst_base = src_base
        else:
            src_base = (cid * NT + sid) * nb
            dst_base = sid * nb
        pltpu.sync_copy(zeros, acc.at[pl.ds(sid * ZPT, ZPT)])
        plsc.subcore_barrier()

        gsems = (gsem0, gsem1)
        ssems = (ssem0, ssem1)

        def blk(b, carry):
            pltpu.sync_copy(srcs.at[src_base + b], src_v)
            pltpu.sync_copy(dsts.at[dst_base + b], dst_v)
            gd = [None, None]
            sd = [None, None]
            gd[0] = pltpu.async_copy(table.at[src_v.at[0]], rows_v.at[0], gsems[0])
            for k in range(KB):
                s = k % 2
                gd[s].wait()
                sd[s] = pltpu.async_copy(rows_v.at[s], acc.at[dst_v.at[k]],
                                         ssems[s], add=True)
                if k + 1 < KB:
                    if sd[1 - s] is not None:
                        sd[1 - s].wait()
                    gd[1 - s] = pltpu.async_copy(
                        table.at[src_v.at[k + 1]], rows_v.at[1 - s], gsems[1 - s])
            sd[(KB - 2) % 2].wait()
            sd[(KB - 1) % 2].wait()
            return carry

        lax.fori_loop(0, nb, blk, 0, unroll=False)
        plsc.subcore_barrier()
        pltpu.sync_copy(acc.at[pl.ds(sid * ZPT, ZPT)],
                        out.at[pl.ds(cid * NP + sid * ZPT, ZPT)])

    return seg_sum


_seg1 = _make_seg_sum(NB1, True)
_seg2 = _make_seg_sum(NB2, False)

_BR = 2528


def _mm1_body(a0, a1, w, o):
    ax = a0[...] + a1[...]
    o[...] = jnp.maximum(jnp.dot(ax, w[...], preferred_element_type=jnp.float32), 0.0)


_NRB = NP // _BR


def _mm1(a0, a1, w0):
    grid = (2, _NRB)
    return pl.pallas_call(
        _mm1_body,
        grid=grid,
        in_specs=[
            pl.BlockSpec((_BR, 128), lambda c, i: (i, 0)),
            pl.BlockSpec((_BR, 128), lambda c, i: (i, 0)),
            pl.BlockSpec((128, 128), lambda c, i: (0, c)),
        ],
        out_specs=pl.BlockSpec((_BR, 128), lambda c, i: (c * _NRB + i, 0)),
        out_shape=jax.ShapeDtypeStruct((2 * NP, 128), jnp.float32),
    )(a0, a1, w0)


def _mm2_body(a0, a1, w, o):
    ah = jnp.concatenate([a0[...], a1[...]], axis=1)
    o[...] = jnp.dot(ah, w[...], preferred_element_type=jnp.float32)


def _mm2(a0, a1, w12):
    grid = (pl.cdiv(N, _BR),)
    return pl.pallas_call(
        _mm2_body,
        grid=grid,
        in_specs=[
            pl.BlockSpec((_BR, 128), lambda i: (i, 0)),
            pl.BlockSpec((_BR, 128), lambda i: (i, 0)),
            pl.BlockSpec((256, 256), lambda i: (0, 0)),
        ],
        out_specs=pl.BlockSpec((_BR, 256), lambda i: (i, 0)),
        out_shape=jax.ShapeDtypeStruct((N, 256), jnp.float32),
    )(a0, a1, w12)


def kernel(x, edge_index, W0, W1, W2):
    src = edge_index[0]
    dst = edge_index[1]
    zeros = jnp.zeros((ZPT, 128), jnp.float32)

    pad = E_PAD - E
    pad_src = jnp.arange(pad, dtype=jnp.int32) % N
    pad_dst = N + (jnp.arange(pad, dtype=jnp.int32) % (NP - N))
    srcp = jnp.concatenate([src, pad_src]).reshape(NBLK, KB, CHUNK)
    dstp = jnp.concatenate([dst, pad_dst]).reshape(NBLK, KB, CHUNK)
    src2 = jnp.concatenate([srcp, srcp + NP], axis=0)

    xp = jnp.concatenate([x, jnp.zeros((NP - N, 128), jnp.float32)], axis=0)
    ax = _seg1(xp, srcp, dstp, zeros)
    h = _mm1(ax[:NP], ax[NP:], W0)
    ah = _seg2(h, src2, dstp, zeros)
    out = _mm2(ah[:NP], ah[NP:], jnp.concatenate([W1, W2], axis=1))
    return out


# speedup vs baseline: 1.0513x; 1.0049x over previous
import functools

import jax
import jax.numpy as jnp
from jax import lax
from jax.experimental import pallas as pl
from jax.experimental.pallas import tpu as pltpu
from jax.experimental.pallas import tpu_sc as plsc

N = 10000
E = 320000
NT = 16
NC = 2
NW = NC * NT
CHUNK = 128
KB = 40
NB1 = 2
NB2 = 4
E_PAD = NW * NB1 * KB * CHUNK
NBLK = E_PAD // (KB * CHUNK)
NP = 10112
ZPT = NP // NT


def _make_seg_sum(nb, split_edges):
    mesh = plsc.VectorSubcoreMesh(core_axis_name="c", subcore_axis_name="s")

    @functools.partial(
        pl.kernel,
        out_type=jax.ShapeDtypeStruct((NC * NP, 128), jnp.float32),
        mesh=mesh,
        scratch_types=[
            pltpu.VMEM((KB, CHUNK), jnp.int32),
            pltpu.VMEM((KB, CHUNK), jnp.int32),
            pltpu.VMEM((2, CHUNK, 128), jnp.float32),
            pltpu.VMEM_SHARED((NP, 128), jnp.float32),
            pltpu.SemaphoreType.DMA,
            pltpu.SemaphoreType.DMA,
            pltpu.SemaphoreType.DMA,
            pltpu.SemaphoreType.DMA,
        ],
    )
    def seg_sum(table, srcs, dsts, zeros, out, src_v, dst_v, rows_v, acc,
                gsem0, gsem1, ssem0, ssem1):
        cid = lax.axis_index("c")
        sid = lax.axis_index("s")
        if split_edges:
            src_base = (cid * NT + sid) * nb
            dst_base = src_base
        else:
            src_base = (cid * NT + sid) * nb
            dst_base = sid * nb
        pltpu.sync_copy(zeros, acc.at[pl.ds(sid * ZPT, ZPT)])
        plsc.subcore_barrier()

        gsems = (gsem0, gsem1)
        ssems = (ssem0, ssem1)

        def blk(b, carry):
            pltpu.sync_copy(srcs.at[src_base + b], src_v)
            pltpu.sync_copy(dsts.at[dst_base + b], dst_v)
            gd = [None, None]
            sd = [None, None]
            gd[0] = pltpu.async_copy(table.at[src_v.at[0]], rows_v.at[0], gsems[0])
            for k in range(KB):
                s = k % 2
                gd[s].wait()
                sd[s] = pltpu.async_copy(rows_v.at[s], acc.at[dst_v.at[k]],
                                         ssems[s], add=True)
                if k + 1 < KB:
                    if sd[1 - s] is not None:
                        sd[1 - s].wait()
                    gd[1 - s] = pltpu.async_copy(
                        table.at[src_v.at[k + 1]], rows_v.at[1 - s], gsems[1 - s])
            sd[(KB - 2) % 2].wait()
            sd[(KB - 1) % 2].wait()
            return carry

        lax.fori_loop(0, nb, blk, 0, unroll=False)
        plsc.subcore_barrier()
        pltpu.sync_copy(acc.at[pl.ds(sid * ZPT, ZPT)],
                        out.at[pl.ds(cid * NP + sid * ZPT, ZPT)])

    return seg_sum


_seg1 = _make_seg_sum(NB1, True)
_seg2 = _make_seg_sum(NB2, False)

_BR = 5056


def _mm1_body(a0, a1, w, o):
    ax = a0[...] + a1[...]
    o[...] = jnp.maximum(jnp.dot(ax, w[...], preferred_element_type=jnp.float32), 0.0)


_NRB = NP // _BR


def _mm1(a0, a1, w0):
    grid = (2, _NRB)
    return pl.pallas_call(
        _mm1_body,
        grid=grid,
        in_specs=[
            pl.BlockSpec((_BR, 128), lambda c, i: (i, 0)),
            pl.BlockSpec((_BR, 128), lambda c, i: (i, 0)),
            pl.BlockSpec((128, 128), lambda c, i: (0, c)),
        ],
        out_specs=pl.BlockSpec((_BR, 128), lambda c, i: (c * _NRB + i, 0)),
        out_shape=jax.ShapeDtypeStruct((2 * NP, 128), jnp.float32),
    )(a0, a1, w0)


def _mm2_body(a0, a1, w, o):
    ah = jnp.concatenate([a0[...], a1[...]], axis=1)
    o[...] = jnp.dot(ah, w[...], preferred_element_type=jnp.float32)


def _mm2(a0, a1, w12):
    grid = (pl.cdiv(N, _BR),)
    return pl.pallas_call(
        _mm2_body,
        grid=grid,
        in_specs=[
            pl.BlockSpec((_BR, 128), lambda i: (i, 0)),
            pl.BlockSpec((_BR, 128), lambda i: (i, 0)),
            pl.BlockSpec((256, 256), lambda i: (0, 0)),
        ],
        out_specs=pl.BlockSpec((_BR, 256), lambda i: (i, 0)),
        out_shape=jax.ShapeDtypeStruct((N, 256), jnp.float32),
    )(a0, a1, w12)


def kernel(x, edge_index, W0, W1, W2):
    src = edge_index[0]
    dst = edge_index[1]
    zeros = jnp.zeros((ZPT, 128), jnp.float32)

    pad = E_PAD - E
    pad_src = jnp.arange(pad, dtype=jnp.int32) % N
    pad_dst = N + (jnp.arange(pad, dtype=jnp.int32) % (NP - N))
    srcp = jnp.concatenate([src, pad_src]).reshape(NBLK, KB, CHUNK)
    dstp = jnp.concatenate([dst, pad_dst]).reshape(NBLK, KB, CHUNK)
    src2 = jnp.concatenate([srcp, srcp + NP], axis=0)

    xp = jnp.concatenate([x, jnp.zeros((NP - N, 128), jnp.float32)], axis=0)
    ax = _seg1(xp, srcp, dstp, zeros)
    h = _mm1(ax[:NP], ax[NP:], W0)
    ah = _seg2(h, src2, dstp, zeros)
    out = _mm2(ah[:NP], ah[NP:], jnp.concatenate([W1, W2], axis=1))
    return out


# speedup vs baseline: 1.0650x; 1.0130x over previous
import functools

import jax
import jax.numpy as jnp
from jax import lax
from jax.experimental import pallas as pl
from jax.experimental.pallas import tpu as pltpu
from jax.experimental.pallas import tpu_sc as plsc

N = 10000
E = 320000
NT = 16
NC = 2
NW = NC * NT
CHUNK = 128
KB = 40
NB1 = 2
NB2 = 4
E_PAD = NW * NB1 * KB * CHUNK
NBLK = E_PAD // (KB * CHUNK)
NP = 10112
ZPT = NP // NT


def _make_seg_sum(nb, split_edges):
    mesh = plsc.VectorSubcoreMesh(core_axis_name="c", subcore_axis_name="s")

    @functools.partial(
        pl.kernel,
        out_type=jax.ShapeDtypeStruct((NC * NP, 128), jnp.float32),
        mesh=mesh,
        scratch_types=[
            pltpu.VMEM((KB, CHUNK), jnp.int32),
            pltpu.VMEM((KB, CHUNK), jnp.int32),
            pltpu.VMEM((2, CHUNK, 128), jnp.float32),
            pltpu.VMEM_SHARED((NP, 128), jnp.float32),
            pltpu.SemaphoreType.DMA,
            pltpu.SemaphoreType.DMA,
            pltpu.SemaphoreType.DMA,
            pltpu.SemaphoreType.DMA,
            pltpu.SemaphoreType.DMA,
            pltpu.SemaphoreType.DMA,
        ],
    )
    def seg_sum(table, srcs, dsts, zeros, out, src_v, dst_v, rows_v, acc,
                gsem0, gsem1, ssem0, ssem1, isem0, isem1):
        cid = lax.axis_index("c")
        sid = lax.axis_index("s")
        if split_edges:
            src_base = (cid * NT + sid) * nb
            dst_base = src_base
        else:
            src_base = (cid * NT + sid) * nb
            dst_base = sid * nb
        pltpu.sync_copy(zeros, acc.at[pl.ds(sid * ZPT, ZPT)])
        plsc.subcore_barrier()

        gsems = (gsem0, gsem1)
        ssems = (ssem0, ssem1)

        def blk(b, carry):
            ld_s = pltpu.async_copy(srcs.at[src_base + b], src_v, isem0)
            ld_d = pltpu.async_copy(dsts.at[dst_base + b], dst_v, isem1)
            gd = [None, None]
            sd = [None, None]
            ld_s.wait()
            gd[0] = pltpu.async_copy(table.at[src_v.at[0]], rows_v.at[0], gsems[0])
            ld_d.wait()
            for k in range(KB):
                s = k % 2
                gd[s].wait()
                sd[s] = pltpu.async_copy(rows_v.at[s], acc.at[dst_v.at[k]],
                                         ssems[s], add=True)
                if k + 1 < KB:
                    if sd[1 - s] is not None:
                        sd[1 - s].wait()
                    gd[1 - s] = pltpu.async_copy(
                        table.at[src_v.at[k + 1]], rows_v.at[1 - s], gsems[1 - s])
            sd[(KB - 2) % 2].wait()
            sd[(KB - 1) % 2].wait()
            return carry

        lax.fori_loop(0, nb, blk, 0, unroll=False)
        plsc.subcore_barrier()
        pltpu.sync_copy(acc.at[pl.ds(sid * ZPT, ZPT)],
                        out.at[pl.ds(cid * NP + sid * ZPT, ZPT)])

    return seg_sum


_seg1 = _make_seg_sum(NB1, True)
_seg2 = _make_seg_sum(NB2, False)

_BR = 5056


def _mm1_body(a0, a1, w, o):
    ax = a0[...] + a1[...]
    o[...] = jnp.maximum(jnp.dot(ax, w[...], preferred_element_type=jnp.float32), 0.0)


_NRB = NP // _BR


def _mm1(a0, a1, w0):
    grid = (2, _NRB)
    return pl.pallas_call(
        _mm1_body,
        grid=grid,
        in_specs=[
            pl.BlockSpec((_BR, 128), lambda c, i: (i, 0)),
            pl.BlockSpec((_BR, 128), lambda c, i: (i, 0)),
            pl.BlockSpec((128, 128), lambda c, i: (0, c)),
        ],
        out_specs=pl.BlockSpec((_BR, 128), lambda c, i: (c * _NRB + i, 0)),
        out_shape=jax.ShapeDtypeStruct((2 * NP, 128), jnp.float32),
    )(a0, a1, w0)


def _mm2_body(a0, a1, w, o):
    ah = jnp.concatenate([a0[...], a1[...]], axis=1)
    o[...] = jnp.dot(ah, w[...], preferred_element_type=jnp.float32)


def _mm2(a0, a1, w12):
    grid = (pl.cdiv(N, _BR),)
    return pl.pallas_call(
        _mm2_body,
        grid=grid,
        in_specs=[
            pl.BlockSpec((_BR, 128), lambda i: (i, 0)),
            pl.BlockSpec((_BR, 128), lambda i: (i, 0)),
            pl.BlockSpec((256, 256), lambda i: (0, 0)),
        ],
        out_specs=pl.BlockSpec((_BR, 256), lambda i: (i, 0)),
        out_shape=jax.ShapeDtypeStruct((N, 256), jnp.float32),
    )(a0, a1, w12)


def kernel(x, edge_index, W0, W1, W2):
    src = edge_index[0]
    dst = edge_index[1]
    zeros = jnp.zeros((ZPT, 128), jnp.float32)

    pad = E_PAD - E
    pad_src = jnp.arange(pad, dtype=jnp.int32) % N
    pad_dst = N + (jnp.arange(pad, dtype=jnp.int32) % (NP - N))
    srcp = jnp.concatenate([src, pad_src]).reshape(NBLK, KB, CHUNK)
    dstp = jnp.concatenate([dst, pad_dst]).reshape(NBLK, KB, CHUNK)
    src2 = jnp.concatenate([srcp, srcp + NP], axis=0)

    xp = jnp.concatenate([x, jnp.zeros((NP - N, 128), jnp.float32)], axis=0)
    ax = _seg1(xp, srcp, dstp, zeros)
    h = _mm1(ax[:NP], ax[NP:], W0)
    ah = _seg2(h, src2, dstp, zeros)
    out = _mm2(ah[:NP], ah[NP:], jnp.concatenate([W1, W2], axis=1))
    return out
